# Initial kernel scaffold; baseline (speedup 1.0000x reference)
import jax
import jax.numpy as jnp
from jax.experimental import pallas as pl


def kernel(x, adj_g, W_emb, b_emb, W_ass, b_ass, att_W, att_b, mlp_W1, mlp_b1, mlp_W2, mlp_b2, edge_index):
    raise NotImplementedError("write your pallas kernel here")


# speedup vs baseline: 1.0001x; 1.0001x over previous
import jax
import jax.numpy as jnp
from jax.experimental import pallas as pl

K = 16


def _knn_adj(z, k):
    z = jax.lax.stop_gradient(z)
    sq = jnp.sum(z * z, axis=1)
    d2 = sq[:, None] + sq[None, :] - 2.0 * (z @ z.T)
    _, idx = jax.lax.top_k(-d2, k)
    n = z.shape[0]
    rows = jnp.repeat(jnp.arange(n), k)
    adj = jnp.zeros((n, n), jnp.float32).at[rows, idx.reshape(-1)].set(1.0)
    return (adj + adj.T) / 2.0


def kernel(x, adj_g, W_emb, b_emb, W_ass, b_ass, att_W, att_b, mlp_W1, mlp_b1, mlp_W2, mlp_b2, edge_index):
    src = edge_index[0]
    dst = edge_index[1]
    n = x.shape[0]
    h_lin = x @ W_emb + b_emb
    ones = jnp.ones((src.shape[0],), jnp.float32)
    deg = jax.ops.segment_sum(ones, dst, num_segments=n)
    h = jax.ops.segment_sum(h_lin[src], dst, num_segments=n) / jnp.maximum(deg, 1.0)[:, None]
    g = x @ W_ass + b_ass
    z2 = jnp.concatenate([g[src], g[dst]], axis=1)
    e_att = jax.nn.leaky_relu(z2 @ att_W + att_b)
    m = jax.ops.segment_max(e_att, dst, num_segments=n)
    ex = jnp.exp(e_att - m[dst])
    denom = jax.ops.segment_sum(ex, dst, num_segments=n)
    alpha = ex / jnp.maximum(denom[dst], 1e-38)
    s_pre = jax.ops.segment_sum(alpha * g[src], dst, num_segments=n)
    s = jax.nn.softmax(s_pre, axis=-1)
    e = s.T @ h
    adj_g1 = s.T @ adj_g @ s
    me = e @ mlp_W1 + mlp_b1
    me = me @ mlp_W2 + mlp_b2
    adj_f1 = _knn_adj(me, K)
    return (h, e, s, adj_g1, adj_f1)


# speedup vs baseline: 4.7803x; 4.7800x over previous
import functools

import jax
import jax.numpy as jnp
from jax import lax
from jax.experimental import pallas as pl
from jax.experimental.pallas import tpu as pltpu
from jax.experimental.pallas import tpu_sc as plsc

N = 10000
D = 256
C = 128
E = 160000
KNN_K = 16

NP = 10112
CH = 128
EP = 163840
NTILES = 16
ROWS_PT = NP // NTILES
NCH = EP // (NTILES * CH)
HG = 64
GW = 64
WW = 80

_mesh = plsc.VectorSubcoreMesh(core_axis_name="c", subcore_axis_name="s")


def _a_body(x_ref, we_ref, be_ref, wa_ref, ba_ref, watt_ref,
            h0_ref, h1_ref, h2_ref, h3_ref, g0_ref, g1_ref, a8_ref):
    x = x_ref[...].astype(jnp.bfloat16)
    hl = jnp.dot(x, we_ref[...].astype(jnp.bfloat16),
                 preferred_element_type=jnp.float32) + be_ref[...]
    g = jnp.dot(x, wa_ref[...].astype(jnp.bfloat16),
                preferred_element_type=jnp.float32) + ba_ref[...]
    a8 = jnp.dot(g.astype(jnp.bfloat16), watt_ref[...].astype(jnp.bfloat16),
                 preferred_element_type=jnp.float32)
    h0_ref[...] = hl[:, 0 * HG:1 * HG]
    h1_ref[...] = hl[:, 1 * HG:2 * HG]
    h2_ref[...] = hl[:, 2 * HG:3 * HG]
    h3_ref[...] = hl[:, 3 * HG:4 * HG]
    g0_ref[...] = g[:, :GW]
    g1_ref[...] = g[:, GW:]
    a8_ref[...] = a8


def _tc_a(x, W_emb, b_emb, W_ass, b_ass, watt8):
    blk = 400
    grid = (N // blk,)
    hspec = pl.BlockSpec((blk, HG), lambda i: (i, 0))
    hshape = jax.ShapeDtypeStruct((N, HG), jnp.float32)
    return pl.pallas_call(
        _a_body,
        grid=grid,
        in_specs=[
            pl.BlockSpec((blk, D), lambda i: (i, 0)),
            pl.BlockSpec((D, D), lambda i: (0, 0)),
            pl.BlockSpec((1, D), lambda i: (0, 0)),
            pl.BlockSpec((D, C), lambda i: (0, 0)),
            pl.BlockSpec((1, C), lambda i: (0, 0)),
            pl.BlockSpec((C, 8), lambda i: (0, 0)),
        ],
        out_specs=[hspec, hspec, hspec, hspec, hspec, hspec,
                   pl.BlockSpec((blk, 8), lambda i: (i, 0))],
        out_shape=[hshape, hshape, hshape, hshape, hshape, hshape,
                   jax.ShapeDtypeStruct((N, 8), jnp.float32)],
    )(x, W_emb, b_emb, W_ass, b_ass, watt8)


def _k1_body(tab0, tab1, srcm, dstm, zer, out0, out1,
             srcb, dstb, rbuf, acc, gsem):
    tid = lax.axis_index("s")
    core = lax.axis_index("c")
    pltpu.sync_copy(srcm.at[pl.ds(tid * NCH, NCH)], srcb)
    pltpu.sync_copy(dstm.at[pl.ds(tid * NCH, NCH)], dstb)
    rows0 = tid * ROWS_PT
    pltpu.sync_copy(zer.at[pl.ds(rows0, ROWS_PT)], acc.at[pl.ds(rows0, ROWS_PT)])
    plsc.subcore_barrier()

    def phase(table, out):
        pltpu.async_copy(table.at[srcb.at[0]], rbuf.at[0], gsem)

        def body(j, carry):
            b = lax.rem(j, 2)
            pltpu.make_async_copy(table.at[srcb.at[j]], rbuf.at[b], gsem).wait()

            @pl.when(j + 1 < NCH)
            def _():
                pltpu.async_copy(table.at[srcb.at[j + 1]], rbuf.at[1 - b], gsem)

            pltpu.sync_copy(rbuf.at[b], acc.at[dstb.at[j]], add=True)
            return carry

        lax.fori_loop(0, NCH, body, 0)
        plsc.subcore_barrier()
        pltpu.sync_copy(acc.at[pl.ds(rows0, ROWS_PT)], out.at[pl.ds(rows0, ROWS_PT)])

    @pl.when(core == 0)
    def _():
        phase(tab0, out0)

    @pl.when(core == 1)
    def _():
        phase(tab1, out1)


def _sc_k1(tab0, tab1, src_mat, dst_mat, zeros_h):
    k = functools.partial(
        pl.kernel,
        mesh=_mesh,
        out_type=[
            jax.ShapeDtypeStruct((NP, HG), jnp.float32),
            jax.ShapeDtypeStruct((NP, HG), jnp.float32),
        ],
        scratch_types=[
            pltpu.VMEM((NCH, CH), jnp.int32),
            pltpu.VMEM((NCH, CH), jnp.int32),
            pltpu.VMEM((2, CH, HG), jnp.float32),
            pltpu.VMEM_SHARED((NP, HG), jnp.float32),
            pltpu.SemaphoreType.DMA,
        ],
        compiler_params=pltpu.CompilerParams(use_tc_tiling_on_sc=False),
    )(_k1_body)
    return k(tab0, tab1, src_mat, dst_mat, zeros_h)


def _k2_body(g0, g1, srcm, dstm, asrc, adst, zer, out,
             srcb, dstb, asb, adb, gbuf, wbuf, acc, gsem):
    tid = lax.axis_index("s")
    core = lax.axis_index("c")
    pltpu.sync_copy(asrc, asb)
    pltpu.sync_copy(adst, adb)
    pltpu.sync_copy(srcm.at[pl.ds(tid * NCH, NCH)], srcb)
    pltpu.sync_copy(dstm.at[pl.ds(tid * NCH, NCH)], dstb)
    rows0 = tid * ROWS_PT
    pltpu.sync_copy(zer.at[pl.ds(rows0, ROWS_PT)], acc.at[pl.ds(rows0, ROWS_PT)])
    plsc.subcore_barrier()

    lane = lax.broadcasted_iota(jnp.int32, (16,), 0)

    def phase(table):
        pltpu.async_copy(table.at[srcb.at[0]], gbuf.at[0], gsem)

        def body(j, carry):
            b = lax.rem(j, 2)
            pltpu.make_async_copy(table.at[srcb.at[j]], gbuf.at[b], gsem).wait()

            @pl.when(j + 1 < NCH)
            def _():
                pltpu.async_copy(table.at[srcb.at[j + 1]], gbuf.at[1 - b], gsem)

            def vloop(v, c2):
                sv = srcb[j, pl.ds(v * 16, 16)]
                dv = dstb[j, pl.ds(v * 16, 16)]
                a_s = plsc.load_gather(asb, [sv])
                a_d = plsc.load_gather(adb, [dv])
                z = a_s + a_d
                e_att = jnp.maximum(z, 0.01 * z)
                exv = jnp.exp(e_att)
                scs = [exv[l] for l in range(16)]
                for k in range(GW // 16):
                    vals = [gbuf[b, v * 16 + l, pl.ds(k * 16, 16)]
                            for l in range(16)]
                    for l in range(16):
                        wbuf[v * 16 + l, pl.ds(k * 16, 16)] = vals[l] * scs[l]
                for l in range(16):
                    e = v * 16 + l
                    tail = jnp.where(lane == 0, scs[l],
                                     jnp.where(lane == 1, jnp.float32(1.0),
                                               jnp.float32(0.0)))
                    wbuf[e, pl.ds(GW, 16)] = tail
                return c2

            lax.fori_loop(0, CH // 16, vloop, 0)
            pltpu.sync_copy(wbuf, acc.at[dstb.at[j]], add=True)
            return carry

        lax.fori_loop(0, NCH, body, 0)

    @pl.when(core == 0)
    def _():
        phase(g0)

    @pl.when(core == 1)
    def _():
        phase(g1)

    plsc.subcore_barrier()
    pltpu.sync_copy(acc.at[pl.ds(rows0, ROWS_PT)],
                    out.at[core].at[pl.ds(rows0, ROWS_PT)])


def _sc_k2(g0, g1, src_mat, dst_mat, a_src, a_dst, zeros_w):
    k = functools.partial(
        pl.kernel,
        mesh=_mesh,
        out_type=[
            jax.ShapeDtypeStruct((2, NP, WW), jnp.float32),
        ],
        scratch_types=[
            pltpu.VMEM((NCH, CH), jnp.int32),
            pltpu.VMEM((NCH, CH), jnp.int32),
            pltpu.VMEM((NP,), jnp.float32),
            pltpu.VMEM((NP,), jnp.float32),
            pltpu.VMEM((2, CH, GW), jnp.float32),
            pltpu.VMEM((CH, WW), jnp.float32),
            pltpu.VMEM_SHARED((NP, WW), jnp.float32),
            pltpu.SemaphoreType.DMA,
        ],
        compiler_params=pltpu.CompilerParams(use_tc_tiling_on_sc=False,
                                             needs_layout_passes=False),
    )(_k2_body)
    return k(g0, g1, src_mat, dst_mat, a_src, a_dst, zeros_w)


def _b_body(w0_ref, w1_ref, h0_ref, h1_ref, h2_ref, h3_ref, adj_ref,
            h_ref, s_ref, t_ref, e_ref):
    i = pl.program_id(0)
    w0 = w0_ref[...]
    w1 = w1_ref[...]
    den = w0[:, GW:GW + 1]
    deg = w0[:, GW + 1:GW + 2]
    s_pre = jnp.concatenate([w0[:, :GW], w1[:, :GW]], axis=1) / jnp.maximum(den, 1e-38)
    sm = s_pre - jnp.max(s_pre, axis=1, keepdims=True)
    p = jnp.exp(sm)
    s = p / jnp.sum(p, axis=1, keepdims=True)
    h = jnp.concatenate(
        [h0_ref[...], h1_ref[...], h2_ref[...], h3_ref[...]], axis=1
    ) / jnp.maximum(deg, 1.0)
    h_ref[...] = h
    s_ref[...] = s

    sb = s.astype(jnp.bfloat16)
    st_adj = lax.dot_general(sb, adj_ref[...].astype(jnp.bfloat16),
                             (((0,), (0,)), ((), ())),
                             preferred_element_type=jnp.float32)
    st_h = lax.dot_general(sb, h.astype(jnp.bfloat16),
                           (((0,), (0,)), ((), ())),
                           preferred_element_type=jnp.float32)

    @pl.when(i == 0)
    def _():
        t_ref[...] = jnp.zeros_like(t_ref)
        e_ref[...] = jnp.zeros_like(e_ref)

    t_ref[...] += st_adj
    e_ref[...] += st_h


def _tc_b(w0, w1, hn0, hn1, hn2, hn3, adj_g):
    blk = 200
    grid = (N // blk,)
    hspec = pl.BlockSpec((blk, HG), lambda i: (i, 0))
    return pl.pallas_call(
        _b_body,
        grid=grid,
        in_specs=[
            pl.BlockSpec((blk, WW), lambda i: (i, 0)),
            pl.BlockSpec((blk, WW), lambda i: (i, 0)),
            hspec, hspec, hspec, hspec,
            pl.BlockSpec((blk, N), lambda i: (i, 0)),
        ],
        out_specs=[
            pl.BlockSpec((blk, D), lambda i: (i, 0)),
            pl.BlockSpec((blk, C), lambda i: (i, 0)),
            pl.BlockSpec((C, N), lambda i: (0, 0)),
            pl.BlockSpec((C, D), lambda i: (0, 0)),
        ],
        out_shape=[
            jax.ShapeDtypeStruct((N, D), jnp.float32),
            jax.ShapeDtypeStruct((N, C), jnp.float32),
            jax.ShapeDtypeStruct((C, N), jnp.float32),
            jax.ShapeDtypeStruct((C, D), jnp.float32),
        ],
        compiler_params=pltpu.CompilerParams(
            dimension_semantics=("arbitrary",),
        ),
    )(w0, w1, hn0, hn1, hn2, hn3, adj_g)


def _c_body(t_ref, s_ref, e_ref, w1_ref, b1_ref, w2_ref, b2_ref,
            adjg1_ref, adjf1_ref):
    adjg1_ref[...] = jnp.dot(t_ref[...].astype(jnp.bfloat16),
                             s_ref[...].astype(jnp.bfloat16),
                             preferred_element_type=jnp.float32)
    me = jnp.dot(e_ref[...].astype(jnp.bfloat16),
                 w1_ref[...].astype(jnp.bfloat16),
                 preferred_element_type=jnp.float32) + b1_ref[...]
    me = jnp.dot(me.astype(jnp.bfloat16), w2_ref[...].astype(jnp.bfloat16),
                 preferred_element_type=jnp.float32) + b2_ref[...]
    sq = jnp.sum(me * me, axis=1, keepdims=True)
    meb = me.astype(jnp.bfloat16)
    cross = lax.dot_general(meb, meb, (((1,), (1,)), ((), ())),
                            preferred_element_type=jnp.float32)
    val = -(sq + sq.T - 2.0 * cross)
    iota_r = lax.broadcasted_iota(jnp.int32, (C, C), 1)
    adj = jnp.zeros((C, C), jnp.float32)
    for _ in range(KNN_K):
        m = jnp.max(val, axis=1, keepdims=True)
        cand = jnp.where(val == m, iota_r, C)
        amin = jnp.min(cand, axis=1, keepdims=True)
        pick = iota_r == amin
        adj = adj + pick.astype(jnp.float32)
        val = jnp.where(pick, -jnp.inf, val)
    adjf1_ref[...] = (adj + adj.T) * 0.5


def _tc_c(t, s, e, mlp_W1, mlp_b1, mlp_W2, mlp_b2):
    return pl.pallas_call(
        _c_body,
        out_shape=[
            jax.ShapeDtypeStruct((C, C), jnp.float32),
            jax.ShapeDtypeStruct((C, C), jnp.float32),
        ],
    )(t, s, e, mlp_W1, mlp_b1, mlp_W2, mlp_b2)


def kernel(x, adj_g, W_emb, b_emb, W_ass, b_ass, att_W, att_b,
           mlp_W1, mlp_b1, mlp_W2, mlp_b2, edge_index):
    src = edge_index[0]
    dst = edge_index[1]

    watt8 = jnp.zeros((C, 8), jnp.float32)
    watt8 = watt8.at[:, 0].set(att_W[:C, 0]).at[:, 1].set(att_W[C:, 0])

    hq0, hq1, hq2, hq3, g0, g1, a8 = _tc_a(
        x, W_emb, b_emb.reshape(1, D), W_ass, b_ass.reshape(1, C), watt8)

    a_src = jnp.pad(a8[:, 0], (0, NP - N))
    a_dst = jnp.pad(a8[:, 1] + att_b[0], (0, NP - N))

    pad = EP - E
    src_p = jnp.concatenate([src, jnp.zeros((pad,), jnp.int32)])
    dst_p = jnp.concatenate([dst, N + (jnp.arange(pad, dtype=jnp.int32) % 16)])
    src_mat = src_p.reshape(EP // CH, CH)
    dst_mat = dst_p.reshape(EP // CH, CH)

    zeros_h = jnp.zeros((NP, HG), jnp.float32)
    zeros_w = jnp.zeros((NP, WW), jnp.float32)

    hn0, hn1 = _sc_k1(hq0, hq1, src_mat, dst_mat, zeros_h)
    hn2, hn3 = _sc_k1(hq2, hq3, src_mat, dst_mat, zeros_h)
    (wacc,) = _sc_k2(g0, g1, src_mat, dst_mat, a_src, a_dst, zeros_w)

    h, s, t, e = _tc_b(wacc[0, :N], wacc[1, :N],
                       hn0[:N], hn1[:N], hn2[:N], hn3[:N], adj_g)
    adj_g1, adj_f1 = _tc_c(t, s, e, mlp_W1, mlp_b1.reshape(1, D),
                           mlp_W2, mlp_b2.reshape(1, D))
    return (h, e, s, adj_g1, adj_f1)
